# Initial kernel scaffold; baseline (speedup 1.0000x reference)
import jax
import jax.numpy as jnp
from jax.experimental import pallas as pl


def kernel(x, label_maps):
    raise NotImplementedError("write your pallas kernel here")


# speedup vs baseline: 4.9559x; 4.9559x over previous
import functools

import jax
import jax.numpy as jnp
from jax.experimental import pallas as pl
from jax.experimental.pallas import tpu as pltpu

K = 256


def _pool_body(nj, x_ref, lab_ref, out_ref, cnt_ref):
    j = pl.program_id(1)

    labs = lab_ref[0]
    kiota = jax.lax.broadcasted_iota(jnp.int32, (K, labs.shape[-1]), 0)
    onehot = labs == kiota
    oh_bf = onehot.astype(jnp.bfloat16)
    xb = x_ref[0].astype(jnp.bfloat16)

    psum = jax.lax.dot_general(
        oh_bf, xb, (((1,), (1,)), ((), ())),
        preferred_element_type=jnp.float32)
    pcnt = jnp.sum(onehot.astype(jnp.float32), axis=1, keepdims=True)

    @pl.when(j == 0)
    def _init():
        out_ref[0] = psum
        cnt_ref[...] = pcnt

    @pl.when(j > 0)
    def _acc():
        out_ref[0] += psum
        cnt_ref[...] += pcnt

    @pl.when(j == nj - 1)
    def _finish():
        out_ref[0] = out_ref[0] / jnp.maximum(cnt_ref[...], 1.0)


def kernel(x, label_maps):
    B, C, H, W = x.shape
    HW = H * W
    chunk = 3584 if HW % 3584 == 0 else HW
    nj = HW // chunk

    x3 = x.reshape(B, C, HW)
    labs = label_maps.reshape(B * nj, 1, chunk)

    out = pl.pallas_call(
        functools.partial(_pool_body, nj),
        grid=(B, nj),
        in_specs=[
            pl.BlockSpec((1, C, chunk), lambda b, j: (b, 0, j)),
            pl.BlockSpec((1, 1, chunk), lambda b, j: (b * nj + j, 0, 0)),
        ],
        out_specs=pl.BlockSpec((1, K, C), lambda b, j: (b, 0, 0)),
        out_shape=jax.ShapeDtypeStruct((B, K, C), jnp.float32),
        scratch_shapes=[pltpu.VMEM((K, 1), jnp.float32)],
        compiler_params=pltpu.CompilerParams(
            dimension_semantics=("parallel", "arbitrary")),
    )(x3, labs)
    return out
